# Initial kernel scaffold; baseline (speedup 1.0000x reference)
import jax
import jax.numpy as jnp
from jax.experimental import pallas as pl


def kernel(x, edge_index, W_in, b_in, W_g0, b_g0, W_g1, b_g1, W_g2, b_g2, bn_gamma, bn_beta, W_s1, b_s1, W_s2, b_s2, W_o1, b_o1, W_o2, b_o2, W_o3, b_o3):
    raise NotImplementedError("write your pallas kernel here")


# speedup vs baseline: 10.6743x; 10.6743x over previous
import functools
import math

import jax
import jax.numpy as jnp
from jax import lax
from jax.experimental import pallas as pl
from jax.experimental.pallas import tpu as pltpu
from jax.experimental.pallas import tpu_sc as plsc

N = 50000
E = 800000
D_IN = 128
DH = 64
HD = DH // 2
EPS = 1e-5

NSUB = 16
NPAD = 51200
STRIPE = NPAD // NSUB
CHUNK = 128
HALF_CHUNKS = (E // 2) // CHUNK
NFPAD = 50048
FSTRIPE = NFPAD // NSUB
ZROWS = 391

ROWS = 1000
GRID = N // ROWS

_mesh = plsc.VectorSubcoreMesh(core_axis_name="c", subcore_axis_name="s")
_sc_params = pltpu.CompilerParams(use_tc_tiling_on_sc=False)


def _zero_fill_1d(buf, n):
    @pl.loop(0, n, step=16)
    def _(i):
        buf[pl.ds(i, 16)] = jnp.zeros((16,), jnp.float32)


def _zero_fill_2d(buf, rows, cols):
    @pl.loop(0, rows)
    def _(r):
        @pl.loop(0, cols, step=16)
        def _(c):
            buf[r, pl.ds(c, 16)] = jnp.zeros((16,), jnp.float32)


def _sc_degrees(edge_index):
    @functools.partial(
        pl.kernel,
        mesh=_mesh,
        compiler_params=_sc_params,
        out_type=(
            jax.ShapeDtypeStruct((2, NPAD), jnp.float32),
            jax.ShapeDtypeStruct((2, NPAD), jnp.float32),
        ),
        scratch_types=[
            pltpu.VMEM((CHUNK,), jnp.int32),
            pltpu.VMEM((CHUNK,), jnp.int32),
            pltpu.VMEM((CHUNK,), jnp.float32),
            pltpu.VMEM((STRIPE,), jnp.float32),
            pltpu.VMEM_SHARED((NPAD,), jnp.float32),
            pltpu.VMEM_SHARED((NPAD,), jnp.float32),
        ],
    )
    def k(ei, din_out, dout_out, sidx, didx, ones, zbuf, acc_in, acc_out):
        cid = lax.axis_index("c")
        sid = lax.axis_index("s")

        @pl.loop(0, CHUNK, step=16)
        def _(i):
            ones[pl.ds(i, 16)] = jnp.ones((16,), jnp.float32)

        _zero_fill_1d(zbuf, STRIPE)
        pltpu.sync_copy(zbuf, acc_in.at[pl.ds(sid * STRIPE, STRIPE)])
        pltpu.sync_copy(zbuf, acc_out.at[pl.ds(sid * STRIPE, STRIPE)])
        plsc.subcore_barrier()

        base_chunk = cid * HALF_CHUNKS
        trips = 195 + (sid < 5).astype(jnp.int32)

        @pl.loop(0, trips)
        def _(i):
            eb = (base_chunk + sid + NSUB * i) * CHUNK
            pltpu.sync_copy(ei.at[0].at[pl.ds(eb, CHUNK)], sidx)
            pltpu.sync_copy(ei.at[1].at[pl.ds(eb, CHUNK)], didx)
            pltpu.sync_copy(ones, acc_out.at[sidx], add=True)
            pltpu.sync_copy(ones, acc_in.at[didx], add=True)

        plsc.subcore_barrier()
        sl = pl.ds(sid * STRIPE, STRIPE)
        pltpu.sync_copy(acc_in.at[sl], din_out.at[cid].at[sl])
        pltpu.sync_copy(acc_out.at[sl], dout_out.at[cid].at[sl])

    return k(edge_index)


def _sc_influence(edge_index, degf):
    @functools.partial(
        pl.kernel,
        mesh=_mesh,
        compiler_params=_sc_params,
        out_type=jax.ShapeDtypeStruct((2, NPAD), jnp.float32),
        scratch_types=[
            pltpu.VMEM((CHUNK,), jnp.int32),
            pltpu.VMEM((CHUNK,), jnp.int32),
            pltpu.VMEM((CHUNK,), jnp.float32),
            pltpu.VMEM((STRIPE,), jnp.float32),
            pltpu.VMEM_SHARED((NPAD,), jnp.float32),
        ],
    )
    def k(ei, dg, infl_out, sidx, didx, vals, zbuf, acc):
        cid = lax.axis_index("c")
        sid = lax.axis_index("s")
        _zero_fill_1d(zbuf, STRIPE)
        pltpu.sync_copy(zbuf, acc.at[pl.ds(sid * STRIPE, STRIPE)])
        plsc.subcore_barrier()

        base_chunk = cid * HALF_CHUNKS
        trips = 195 + (sid < 5).astype(jnp.int32)

        @pl.loop(0, trips)
        def _(i):
            eb = (base_chunk + sid + NSUB * i) * CHUNK
            pltpu.sync_copy(ei.at[0].at[pl.ds(eb, CHUNK)], sidx)
            pltpu.sync_copy(ei.at[1].at[pl.ds(eb, CHUNK)], didx)
            pltpu.sync_copy(dg.at[didx], vals)
            pltpu.sync_copy(vals, acc.at[sidx], add=True)

        plsc.subcore_barrier()
        sl = pl.ds(sid * STRIPE, STRIPE)
        pltpu.sync_copy(acc.at[sl], infl_out.at[cid].at[sl])

    return k(edge_index, degf)


def _sc_scatter_features(edge_index, hws_a, hws_b):
    @functools.partial(
        pl.kernel,
        mesh=_mesh,
        compiler_params=_sc_params,
        out_type=jax.ShapeDtypeStruct((2, NFPAD, HD), jnp.float32),
        scratch_types=[
            pltpu.VMEM((CHUNK,), jnp.int32),
            pltpu.VMEM((CHUNK,), jnp.int32),
            pltpu.VMEM((CHUNK, HD), jnp.float32),
            pltpu.VMEM((ZROWS, HD), jnp.float32),
            pltpu.VMEM_SHARED((NFPAD, HD), jnp.float32),
        ],
    )
    def k(ei, ha, hb, acc_out, sidx, didx, rows, zbuf, acc):
        cid = lax.axis_index("c")
        sid = lax.axis_index("s")
        _zero_fill_2d(zbuf, ZROWS, HD)

        @pl.loop(0, FSTRIPE, step=ZROWS)
        def _(r):
            pltpu.sync_copy(zbuf, acc.at[pl.ds(sid * FSTRIPE + r, ZROWS)])

        plsc.subcore_barrier()

        trips = 390 + (sid < 10).astype(jnp.int32)

        def edge_loop(table):
            @pl.loop(0, trips)
            def _(i):
                eb = (sid + NSUB * i) * CHUNK
                pltpu.sync_copy(ei.at[0].at[pl.ds(eb, CHUNK)], sidx)
                pltpu.sync_copy(ei.at[1].at[pl.ds(eb, CHUNK)], didx)
                pltpu.sync_copy(table.at[sidx], rows)
                pltpu.sync_copy(rows, acc.at[didx], add=True)

        @pl.when(cid == 0)
        def _():
            edge_loop(ha)

        @pl.when(cid == 1)
        def _():
            edge_loop(hb)

        plsc.subcore_barrier()
        sl = pl.ds(sid * FSTRIPE, FSTRIPE)
        pltpu.sync_copy(acc.at[sl], acc_out.at[cid].at[sl])

    return k(edge_index, hws_a, hws_b)


_INV_BN = 1.0 / math.sqrt(1.0 + EPS)


def _tc_prep(din_parts, dout_parts):
    def body(din, dout, dinv_ref, degf_ref):
        deg = din[0:1, :] + din[1:2, :] + 1.0
        dinv_ref[...] = lax.rsqrt(deg)
        degf_ref[...] = dout[0:1, :] + dout[1:2, :]

    return pl.pallas_call(
        body,
        out_shape=(
            jax.ShapeDtypeStruct((1, NPAD), jnp.float32),
            jax.ShapeDtypeStruct((1, NPAD), jnp.float32),
        ),
    )(din_parts, dout_parts)


def _tc_structural(degf, infl_parts):
    def body(dg, ip, nd_ref, inf_ref):
        dout = dg[...]
        infl_sum = ip[0:1, :] + ip[1:2, :]
        maxd = jnp.max(dout)
        nd_ref[...] = jnp.where(maxd > 0, dout / jnp.maximum(maxd, 1e-12), dout)
        influence = jnp.where(dout > 0, infl_sum / jnp.maximum(dout, 1.0), 0.0)
        maxi = jnp.max(influence)
        inf_ref[...] = jnp.where(
            maxi > 0, influence / jnp.maximum(maxi, 1e-12), influence
        )

    return pl.pallas_call(
        body,
        out_shape=(
            jax.ShapeDtypeStruct((1, NPAD), jnp.float32),
            jax.ShapeDtypeStruct((1, NPAD), jnp.float32),
        ),
    )(degf, infl_parts)


def _split_out(p, out_ref):
    out_ref[0, :, :] = p[:, :HD]
    out_ref[1, :, :] = p[:, HD:]


def _tc_input_layer(x, W_in, b_in, W_g0, dinv_col):
    def body(x_ref, wi, bi, wg, dv, out_ref):
        h = jnp.dot(x_ref[...], wi[...], preferred_element_type=jnp.float32)
        h = h + bi[...]
        p = jnp.dot(h, wg[...], preferred_element_type=jnp.float32) * dv[...]
        _split_out(p, out_ref)

    return pl.pallas_call(
        body,
        grid=(GRID,),
        in_specs=[
            pl.BlockSpec((ROWS, D_IN), lambda i: (i, 0)),
            pl.BlockSpec((D_IN, DH), lambda i: (0, 0)),
            pl.BlockSpec((1, DH), lambda i: (0, 0)),
            pl.BlockSpec((DH, DH), lambda i: (0, 0)),
            pl.BlockSpec((ROWS, 1), lambda i: (i, 0)),
        ],
        out_specs=pl.BlockSpec((2, ROWS, HD), lambda i: (0, i, 0)),
        out_shape=jax.ShapeDtypeStruct((2, N, HD), jnp.float32),
    )(x, W_in, b_in, W_g0, dinv_col)


def _finish_layer(acc, hws, dv, b, g, be):
    m = jnp.concatenate([acc[0] + hws[0], acc[1] + hws[1]], axis=1)
    hn = dv * m + b
    return jnp.maximum((hn * _INV_BN) * g + be, 0.0)


def _tc_mid_layer(acc, hws, dinv_col, b, g, be, W_next):
    def body(acc_ref, hws_ref, dv_ref, b_ref, g_ref, be_ref, w_ref, out_ref):
        h = _finish_layer(
            acc_ref[...], hws_ref[...], dv_ref[...], b_ref[...], g_ref[...], be_ref[...]
        )
        p = jnp.dot(h, w_ref[...], preferred_element_type=jnp.float32) * dv_ref[...]
        _split_out(p, out_ref)

    return pl.pallas_call(
        body,
        grid=(GRID,),
        in_specs=[
            pl.BlockSpec((2, ROWS, HD), lambda i: (0, i, 0)),
            pl.BlockSpec((2, ROWS, HD), lambda i: (0, i, 0)),
            pl.BlockSpec((ROWS, 1), lambda i: (i, 0)),
            pl.BlockSpec((1, DH), lambda i: (0, 0)),
            pl.BlockSpec((1, DH), lambda i: (0, 0)),
            pl.BlockSpec((1, DH), lambda i: (0, 0)),
            pl.BlockSpec((DH, DH), lambda i: (0, 0)),
        ],
        out_specs=pl.BlockSpec((2, ROWS, HD), lambda i: (0, i, 0)),
        out_shape=jax.ShapeDtypeStruct((2, N, HD), jnp.float32),
    )(acc, hws, dinv_col, b, g, be, W_next)


def _tc_output(acc, hws, dinv_col, b, g, be, nd_col, inf_col, W_s1, b_s1, W_s2,
               b_s2, W_o1, b_o1, W_o2, b_o2, W_o3, b_o3):
    def body(acc_ref, hws_ref, dv_ref, b_ref, g_ref, be_ref, nd_ref, inf_ref,
             ws1, bs1, ws2, bs2, wo1, bo1, wo2, bo2, wo3, bo3, out_ref):
        h = _finish_layer(
            acc_ref[...], hws_ref[...], dv_ref[...], b_ref[...], g_ref[...], be_ref[...]
        )
        s_pre = nd_ref[...] * ws1[0:1, :] + inf_ref[...] * ws1[2:3, :] + bs1[...]
        se = jnp.dot(jnp.maximum(s_pre, 0.0), ws2[...],
                     preferred_element_type=jnp.float32) + bs2[...]
        hcat = jnp.concatenate([h, se], axis=1)
        o = jnp.dot(hcat, wo1[...], preferred_element_type=jnp.float32) + bo1[...]
        o = jnp.maximum(o, 0.0)
        o = jnp.dot(o, wo2[...], preferred_element_type=jnp.float32) + bo2[...]
        o = jnp.maximum(o, 0.0)
        o = jnp.dot(o, wo3[...], preferred_element_type=jnp.float32) + bo3[...]
        out_ref[...] = jax.nn.sigmoid(o)

    return pl.pallas_call(
        body,
        grid=(GRID,),
        in_specs=[
            pl.BlockSpec((2, ROWS, HD), lambda i: (0, i, 0)),
            pl.BlockSpec((2, ROWS, HD), lambda i: (0, i, 0)),
            pl.BlockSpec((ROWS, 1), lambda i: (i, 0)),
            pl.BlockSpec((1, DH), lambda i: (0, 0)),
            pl.BlockSpec((1, DH), lambda i: (0, 0)),
            pl.BlockSpec((1, DH), lambda i: (0, 0)),
            pl.BlockSpec((ROWS, 1), lambda i: (i, 0)),
            pl.BlockSpec((ROWS, 1), lambda i: (i, 0)),
            pl.BlockSpec((3, HD), lambda i: (0, 0)),
            pl.BlockSpec((1, HD), lambda i: (0, 0)),
            pl.BlockSpec((HD, DH), lambda i: (0, 0)),
            pl.BlockSpec((1, DH), lambda i: (0, 0)),
            pl.BlockSpec((2 * DH, DH), lambda i: (0, 0)),
            pl.BlockSpec((1, DH), lambda i: (0, 0)),
            pl.BlockSpec((DH, HD), lambda i: (0, 0)),
            pl.BlockSpec((1, HD), lambda i: (0, 0)),
            pl.BlockSpec((HD, 1), lambda i: (0, 0)),
            pl.BlockSpec((1, 1), lambda i: (0, 0)),
        ],
        out_specs=pl.BlockSpec((ROWS, 1), lambda i: (i, 0)),
        out_shape=jax.ShapeDtypeStruct((N, 1), jnp.float32),
    )(acc, hws, dinv_col, b, g, be, nd_col, inf_col, W_s1, b_s1, W_s2, b_s2,
      W_o1, b_o1, W_o2, b_o2, W_o3, b_o3)


def kernel(x, edge_index, W_in, b_in, W_g0, b_g0, W_g1, b_g1, W_g2, b_g2,
           bn_gamma, bn_beta, W_s1, b_s1, W_s2, b_s2, W_o1, b_o1, W_o2, b_o2,
           W_o3, b_o3):
    din_parts, dout_parts = _sc_degrees(edge_index)
    dinv_row, degf_row = _tc_prep(din_parts, dout_parts)
    degf = degf_row.reshape(NPAD)
    infl_parts = _sc_influence(edge_index, degf)
    nd_row, inf_row = _tc_structural(degf_row, infl_parts)

    dinv_col = dinv_row.reshape(NPAD, 1)[:N]
    nd_col = nd_row.reshape(NPAD, 1)[:N]
    inf_col = inf_row.reshape(NPAD, 1)[:N]

    b_in2 = b_in.reshape(1, DH)
    bs = (b_g0.reshape(1, DH), b_g1.reshape(1, DH), b_g2.reshape(1, DH))
    gs = tuple(bn_gamma[i].reshape(1, DH) for i in range(3))
    bes = tuple(bn_beta[i].reshape(1, DH) for i in range(3))

    hws = _tc_input_layer(x, W_in, b_in2, W_g0, dinv_col)
    acc = _sc_scatter_features(edge_index, hws[0], hws[1])
    hws1 = _tc_mid_layer(acc, hws, dinv_col, bs[0], gs[0], bes[0], W_g1)
    acc1 = _sc_scatter_features(edge_index, hws1[0], hws1[1])
    hws2 = _tc_mid_layer(acc1, hws1, dinv_col, bs[1], gs[1], bes[1], W_g2)
    acc2 = _sc_scatter_features(edge_index, hws2[0], hws2[1])

    return _tc_output(
        acc2, hws2, dinv_col, bs[2], gs[2], bes[2], nd_col, inf_col,
        W_s1, b_s1.reshape(1, HD), W_s2, b_s2.reshape(1, DH),
        W_o1, b_o1.reshape(1, DH), W_o2, b_o2.reshape(1, HD),
        W_o3, b_o3.reshape(1, 1),
    )


# speedup vs baseline: 16.0329x; 1.5020x over previous
import functools
import math

import jax
import jax.numpy as jnp
from jax import lax
from jax.experimental import pallas as pl
from jax.experimental.pallas import tpu as pltpu
from jax.experimental.pallas import tpu_sc as plsc

N = 50000
E = 800000
D_IN = 128
DH = 64
HD = DH // 2
EPS = 1e-5

NSUB = 16
NPAD = 51200
STRIPE = NPAD // NSUB
CHUNK = 128
HALF_CHUNKS = (E // 2) // CHUNK
HTRIPS = 195
FCHUNK = 256
ALL_CHUNKS = E // FCHUNK
FTRIPS = 195
NFPAD = 50048
FSTRIPE = NFPAD // NSUB
ZROWS = 184

ROWS = 1000
GRID = N // ROWS

_mesh = plsc.VectorSubcoreMesh(core_axis_name="c", subcore_axis_name="s")
_sc_params = pltpu.CompilerParams(use_tc_tiling_on_sc=False)


def _zero_fill_1d(buf, n):
    @pl.loop(0, n, step=16)
    def _(i):
        buf[pl.ds(i, 16)] = jnp.zeros((16,), jnp.float32)


def _zero_fill_2d(buf, rows, cols):
    @pl.loop(0, rows)
    def _(r):
        @pl.loop(0, cols, step=16)
        def _(c):
            buf[r, pl.ds(c, 16)] = jnp.zeros((16,), jnp.float32)


def _sc_degrees(edge_index):
    @functools.partial(
        pl.kernel,
        mesh=_mesh,
        compiler_params=_sc_params,
        out_type=(
            jax.ShapeDtypeStruct((2, NPAD), jnp.float32),
            jax.ShapeDtypeStruct((2, NPAD), jnp.float32),
        ),
        scratch_types=[
            pltpu.VMEM((2, CHUNK), jnp.int32),
            pltpu.VMEM((2, CHUNK), jnp.int32),
            pltpu.VMEM((CHUNK,), jnp.float32),
            pltpu.VMEM((STRIPE,), jnp.float32),
            pltpu.VMEM_SHARED((NPAD,), jnp.float32),
            pltpu.VMEM_SHARED((NPAD,), jnp.float32),
            pltpu.SemaphoreType.DMA,
            pltpu.SemaphoreType.DMA,
            pltpu.SemaphoreType.DMA,
            pltpu.SemaphoreType.DMA,
        ],
    )
    def k(ei, din_out, dout_out, sidx, didx, ones, zbuf, acc_in, acc_out,
          sem_si0, sem_si1, sem_so0, sem_so1):
        cid = lax.axis_index("c")
        sid = lax.axis_index("s")

        @pl.loop(0, CHUNK // 16)
        def _(i):
            ones[pl.ds(i * 16, 16)] = jnp.ones((16,), jnp.float32)

        _zero_fill_1d(zbuf, STRIPE)
        pltpu.sync_copy(zbuf, acc_in.at[pl.ds(sid * STRIPE, STRIPE)])
        pltpu.sync_copy(zbuf, acc_out.at[pl.ds(sid * STRIPE, STRIPE)])
        plsc.subcore_barrier()

        base_chunk = cid * HALF_CHUNKS
        trips = HTRIPS + (sid < 5).astype(jnp.int32)

        @pl.loop(0, (HTRIPS + 2) // 2)
        def _(g):
            for b in range(2):
                c = 2 * g + b
                sem_si = (sem_si0, sem_si1)[b]
                sem_so = (sem_so0, sem_so1)[b]
                sb, db = sidx.at[b], didx.at[b]

                @pl.when(c < trips)
                def _():
                    @pl.when(c >= 2)
                    def _():
                        pltpu.make_async_copy(ones, acc_out.at[sb], sem_so).wait()
                        pltpu.make_async_copy(ones, acc_in.at[db], sem_si).wait()

                    eb = (base_chunk + sid + NSUB * c) * CHUNK
                    pltpu.sync_copy(ei.at[0].at[pl.ds(eb, CHUNK)], sb)
                    pltpu.sync_copy(ei.at[1].at[pl.ds(eb, CHUNK)], db)
                    pltpu.async_copy(ones, acc_out.at[sb], sem_so, add=True)
                    pltpu.async_copy(ones, acc_in.at[db], sem_si, add=True)

        for b in range(2):
            sem_si = (sem_si0, sem_si1)[b]
            sem_so = (sem_so0, sem_so1)[b]
            pltpu.make_async_copy(ones, acc_out.at[sidx.at[b]], sem_so).wait()
            pltpu.make_async_copy(ones, acc_in.at[didx.at[b]], sem_si).wait()

        plsc.subcore_barrier()
        sl = pl.ds(sid * STRIPE, STRIPE)
        pltpu.sync_copy(acc_in.at[sl], din_out.at[cid].at[sl])
        pltpu.sync_copy(acc_out.at[sl], dout_out.at[cid].at[sl])

    return k(edge_index)


def _sc_influence(edge_index, degf):
    @functools.partial(
        pl.kernel,
        mesh=_mesh,
        compiler_params=_sc_params,
        out_type=jax.ShapeDtypeStruct((2, NPAD), jnp.float32),
        scratch_types=[
            pltpu.VMEM((2, CHUNK), jnp.int32),
            pltpu.VMEM((2, CHUNK), jnp.int32),
            pltpu.VMEM((2, CHUNK), jnp.float32),
            pltpu.VMEM((STRIPE,), jnp.float32),
            pltpu.VMEM_SHARED((NPAD,), jnp.float32),
            pltpu.SemaphoreType.DMA,
            pltpu.SemaphoreType.DMA,
            pltpu.SemaphoreType.DMA,
        ],
    )
    def k(ei, dg, infl_out, sidx, didx, vals, zbuf, acc, sem_g, sem_s0, sem_s1):
        cid = lax.axis_index("c")
        sid = lax.axis_index("s")
        _zero_fill_1d(zbuf, STRIPE)
        pltpu.sync_copy(zbuf, acc.at[pl.ds(sid * STRIPE, STRIPE)])
        plsc.subcore_barrier()

        base_chunk = cid * HALF_CHUNKS
        trips = HTRIPS + (sid < 5).astype(jnp.int32)

        @pl.loop(0, (HTRIPS + 2) // 2)
        def _(g):
            for b in range(2):
                c = 2 * g + b
                sem_s = (sem_s0, sem_s1)[b]
                sb, db, vb = sidx.at[b], didx.at[b], vals.at[b]

                @pl.when(c < trips)
                def _():
                    @pl.when(c >= 2)
                    def _():
                        pltpu.make_async_copy(vb, acc.at[sb], sem_s).wait()

                    eb = (base_chunk + sid + NSUB * c) * CHUNK
                    pltpu.sync_copy(ei.at[0].at[pl.ds(eb, CHUNK)], sb)
                    pltpu.sync_copy(ei.at[1].at[pl.ds(eb, CHUNK)], db)
                    pltpu.async_copy(dg.at[db], vb, sem_g).wait()
                    pltpu.async_copy(vb, acc.at[sb], sem_s, add=True)

        for b in range(2):
            sem_s = (sem_s0, sem_s1)[b]
            pltpu.make_async_copy(vals.at[b], acc.at[sidx.at[b]], sem_s).wait()

        plsc.subcore_barrier()
        sl = pl.ds(sid * STRIPE, STRIPE)
        pltpu.sync_copy(acc.at[sl], infl_out.at[cid].at[sl])

    return k(edge_index, degf)


def _sc_scatter_features(edge_index, hws_a, hws_b):
    @functools.partial(
        pl.kernel,
        mesh=_mesh,
        compiler_params=_sc_params,
        out_type=jax.ShapeDtypeStruct((2, NFPAD, HD), jnp.float32),
        scratch_types=[
            pltpu.VMEM((2, FCHUNK), jnp.int32),
            pltpu.VMEM((2, FCHUNK), jnp.int32),
            pltpu.VMEM((2, FCHUNK, HD), jnp.float32),
            pltpu.VMEM((ZROWS, HD), jnp.float32),
            pltpu.VMEM_SHARED((NFPAD, HD), jnp.float32),
            pltpu.SemaphoreType.DMA,
            pltpu.SemaphoreType.DMA,
            pltpu.SemaphoreType.DMA,
        ],
    )
    def k(ei, ha, hb, acc_out, sidx, didx, rows, zbuf, acc, sem_g, sem_s0,
          sem_s1):
        cid = lax.axis_index("c")
        sid = lax.axis_index("s")
        _zero_fill_2d(zbuf, ZROWS, HD)

        @pl.loop(0, FSTRIPE // ZROWS)
        def _(r):
            pltpu.sync_copy(zbuf, acc.at[pl.ds(sid * FSTRIPE + r * ZROWS, ZROWS)])

        plsc.subcore_barrier()

        trips = FTRIPS + (sid < 5).astype(jnp.int32)

        def edge_loop(table):
            @pl.loop(0, (FTRIPS + 2) // 2)
            def _(g):
                for b in range(2):
                    c = 2 * g + b
                    sem_s = (sem_s0, sem_s1)[b]
                    sb, db, rb = sidx.at[b], didx.at[b], rows.at[b]

                    @pl.when(c < trips)
                    def _():
                        @pl.when(c >= 2)
                        def _():
                            pltpu.make_async_copy(rb, acc.at[db], sem_s).wait()

                        eb = (sid + NSUB * c) * FCHUNK
                        pltpu.sync_copy(ei.at[0].at[pl.ds(eb, FCHUNK)], sb)
                        pltpu.sync_copy(ei.at[1].at[pl.ds(eb, FCHUNK)], db)
                        pltpu.async_copy(table.at[sb], rb, sem_g).wait()
                        pltpu.async_copy(rb, acc.at[db], sem_s, add=True)

            for b in range(2):
                sem_s = (sem_s0, sem_s1)[b]
                pltpu.make_async_copy(rows.at[b], acc.at[didx.at[b]], sem_s).wait()

        @pl.when(cid == 0)
        def _():
            edge_loop(ha)

        @pl.when(cid == 1)
        def _():
            edge_loop(hb)

        plsc.subcore_barrier()
        sl = pl.ds(sid * FSTRIPE, FSTRIPE)
        pltpu.sync_copy(acc.at[sl], acc_out.at[cid].at[sl])

    return k(edge_index, hws_a, hws_b)


_INV_BN = 1.0 / math.sqrt(1.0 + EPS)


def _tc_prep(din_parts, dout_parts):
    def body(din, dout, dinv_ref, degf_ref):
        deg = din[0:1, :] + din[1:2, :] + 1.0
        dinv_ref[...] = lax.rsqrt(deg)
        degf_ref[...] = dout[0:1, :] + dout[1:2, :]

    return pl.pallas_call(
        body,
        out_shape=(
            jax.ShapeDtypeStruct((1, NPAD), jnp.float32),
            jax.ShapeDtypeStruct((1, NPAD), jnp.float32),
        ),
    )(din_parts, dout_parts)


def _tc_structural(degf, infl_parts):
    def body(dg, ip, nd_ref, inf_ref):
        dout = dg[...]
        infl_sum = ip[0:1, :] + ip[1:2, :]
        maxd = jnp.max(dout)
        nd_ref[...] = jnp.where(maxd > 0, dout / jnp.maximum(maxd, 1e-12), dout)
        influence = jnp.where(dout > 0, infl_sum / jnp.maximum(dout, 1.0), 0.0)
        maxi = jnp.max(influence)
        inf_ref[...] = jnp.where(
            maxi > 0, influence / jnp.maximum(maxi, 1e-12), influence
        )

    return pl.pallas_call(
        body,
        out_shape=(
            jax.ShapeDtypeStruct((1, NPAD), jnp.float32),
            jax.ShapeDtypeStruct((1, NPAD), jnp.float32),
        ),
    )(degf, infl_parts)


def _split_out(p, out_ref):
    out_ref[0, :, :] = p[:, :HD]
    out_ref[1, :, :] = p[:, HD:]


def _tc_input_layer(x, W_in, b_in, W_g0, dinv_col):
    def body(x_ref, wi, bi, wg, dv, out_ref):
        h = jnp.dot(x_ref[...], wi[...], preferred_element_type=jnp.float32)
        h = h + bi[...]
        p = jnp.dot(h, wg[...], preferred_element_type=jnp.float32) * dv[...]
        _split_out(p, out_ref)

    return pl.pallas_call(
        body,
        grid=(GRID,),
        in_specs=[
            pl.BlockSpec((ROWS, D_IN), lambda i: (i, 0)),
            pl.BlockSpec((D_IN, DH), lambda i: (0, 0)),
            pl.BlockSpec((1, DH), lambda i: (0, 0)),
            pl.BlockSpec((DH, DH), lambda i: (0, 0)),
            pl.BlockSpec((ROWS, 1), lambda i: (i, 0)),
        ],
        out_specs=pl.BlockSpec((2, ROWS, HD), lambda i: (0, i, 0)),
        out_shape=jax.ShapeDtypeStruct((2, N, HD), jnp.float32),
    )(x, W_in, b_in, W_g0, dinv_col)


def _finish_layer(acc, hws, dv, b, g, be):
    m = jnp.concatenate([acc[0] + hws[0], acc[1] + hws[1]], axis=1)
    hn = dv * m + b
    return jnp.maximum((hn * _INV_BN) * g + be, 0.0)


def _tc_mid_layer(acc, hws, dinv_col, b, g, be, W_next):
    def body(acc_ref, hws_ref, dv_ref, b_ref, g_ref, be_ref, w_ref, out_ref):
        h = _finish_layer(
            acc_ref[...], hws_ref[...], dv_ref[...], b_ref[...], g_ref[...], be_ref[...]
        )
        p = jnp.dot(h, w_ref[...], preferred_element_type=jnp.float32) * dv_ref[...]
        _split_out(p, out_ref)

    return pl.pallas_call(
        body,
        grid=(GRID,),
        in_specs=[
            pl.BlockSpec((2, ROWS, HD), lambda i: (0, i, 0)),
            pl.BlockSpec((2, ROWS, HD), lambda i: (0, i, 0)),
            pl.BlockSpec((ROWS, 1), lambda i: (i, 0)),
            pl.BlockSpec((1, DH), lambda i: (0, 0)),
            pl.BlockSpec((1, DH), lambda i: (0, 0)),
            pl.BlockSpec((1, DH), lambda i: (0, 0)),
            pl.BlockSpec((DH, DH), lambda i: (0, 0)),
        ],
        out_specs=pl.BlockSpec((2, ROWS, HD), lambda i: (0, i, 0)),
        out_shape=jax.ShapeDtypeStruct((2, N, HD), jnp.float32),
    )(acc, hws, dinv_col, b, g, be, W_next)


def _tc_output(acc, hws, dinv_col, b, g, be, nd_col, inf_col, W_s1, b_s1, W_s2,
               b_s2, W_o1, b_o1, W_o2, b_o2, W_o3, b_o3):
    def body(acc_ref, hws_ref, dv_ref, b_ref, g_ref, be_ref, nd_ref, inf_ref,
             ws1, bs1, ws2, bs2, wo1, bo1, wo2, bo2, wo3, bo3, out_ref):
        h = _finish_layer(
            acc_ref[...], hws_ref[...], dv_ref[...], b_ref[...], g_ref[...], be_ref[...]
        )
        s_pre = nd_ref[...] * ws1[0:1, :] + inf_ref[...] * ws1[2:3, :] + bs1[...]
        se = jnp.dot(jnp.maximum(s_pre, 0.0), ws2[...],
                     preferred_element_type=jnp.float32) + bs2[...]
        hcat = jnp.concatenate([h, se], axis=1)
        o = jnp.dot(hcat, wo1[...], preferred_element_type=jnp.float32) + bo1[...]
        o = jnp.maximum(o, 0.0)
        o = jnp.dot(o, wo2[...], preferred_element_type=jnp.float32) + bo2[...]
        o = jnp.maximum(o, 0.0)
        o = jnp.dot(o, wo3[...], preferred_element_type=jnp.float32) + bo3[...]
        out_ref[...] = jax.nn.sigmoid(o)

    return pl.pallas_call(
        body,
        grid=(GRID,),
        in_specs=[
            pl.BlockSpec((2, ROWS, HD), lambda i: (0, i, 0)),
            pl.BlockSpec((2, ROWS, HD), lambda i: (0, i, 0)),
            pl.BlockSpec((ROWS, 1), lambda i: (i, 0)),
            pl.BlockSpec((1, DH), lambda i: (0, 0)),
            pl.BlockSpec((1, DH), lambda i: (0, 0)),
            pl.BlockSpec((1, DH), lambda i: (0, 0)),
            pl.BlockSpec((ROWS, 1), lambda i: (i, 0)),
            pl.BlockSpec((ROWS, 1), lambda i: (i, 0)),
            pl.BlockSpec((3, HD), lambda i: (0, 0)),
            pl.BlockSpec((1, HD), lambda i: (0, 0)),
            pl.BlockSpec((HD, DH), lambda i: (0, 0)),
            pl.BlockSpec((1, DH), lambda i: (0, 0)),
            pl.BlockSpec((2 * DH, DH), lambda i: (0, 0)),
            pl.BlockSpec((1, DH), lambda i: (0, 0)),
            pl.BlockSpec((DH, HD), lambda i: (0, 0)),
            pl.BlockSpec((1, HD), lambda i: (0, 0)),
            pl.BlockSpec((HD, 1), lambda i: (0, 0)),
            pl.BlockSpec((1, 1), lambda i: (0, 0)),
        ],
        out_specs=pl.BlockSpec((ROWS, 1), lambda i: (i, 0)),
        out_shape=jax.ShapeDtypeStruct((N, 1), jnp.float32),
    )(acc, hws, dinv_col, b, g, be, nd_col, inf_col, W_s1, b_s1, W_s2, b_s2,
      W_o1, b_o1, W_o2, b_o2, W_o3, b_o3)


def kernel(x, edge_index, W_in, b_in, W_g0, b_g0, W_g1, b_g1, W_g2, b_g2,
           bn_gamma, bn_beta, W_s1, b_s1, W_s2, b_s2, W_o1, b_o1, W_o2, b_o2,
           W_o3, b_o3):
    din_parts, dout_parts = _sc_degrees(edge_index)
    dinv_row, degf_row = _tc_prep(din_parts, dout_parts)
    degf = degf_row.reshape(NPAD)
    infl_parts = _sc_influence(edge_index, degf)
    nd_row, inf_row = _tc_structural(degf_row, infl_parts)

    dinv_col = dinv_row.reshape(NPAD, 1)[:N]
    nd_col = nd_row.reshape(NPAD, 1)[:N]
    inf_col = inf_row.reshape(NPAD, 1)[:N]

    b_in2 = b_in.reshape(1, DH)
    bs = (b_g0.reshape(1, DH), b_g1.reshape(1, DH), b_g2.reshape(1, DH))
    gs = tuple(bn_gamma[i].reshape(1, DH) for i in range(3))
    bes = tuple(bn_beta[i].reshape(1, DH) for i in range(3))

    hws = _tc_input_layer(x, W_in, b_in2, W_g0, dinv_col)
    acc = _sc_scatter_features(edge_index, hws[0], hws[1])
    hws1 = _tc_mid_layer(acc, hws, dinv_col, bs[0], gs[0], bes[0], W_g1)
    acc1 = _sc_scatter_features(edge_index, hws1[0], hws1[1])
    hws2 = _tc_mid_layer(acc1, hws1, dinv_col, bs[1], gs[1], bes[1], W_g2)
    acc2 = _sc_scatter_features(edge_index, hws2[0], hws2[1])

    return _tc_output(
        acc2, hws2, dinv_col, bs[2], gs[2], bes[2], nd_col, inf_col,
        W_s1, b_s1.reshape(1, HD), W_s2, b_s2.reshape(1, DH),
        W_o1, b_o1.reshape(1, DH), W_o2, b_o2.reshape(1, HD),
        W_o3, b_o3.reshape(1, 1),
    )


# speedup vs baseline: 27.6055x; 1.7218x over previous
import functools
import math

import jax
import jax.numpy as jnp
from jax import lax
from jax.experimental import pallas as pl
from jax.experimental.pallas import tpu as pltpu
from jax.experimental.pallas import tpu_sc as plsc

N = 50000
E = 800000
D_IN = 128
DH = 64
HD = DH // 2
EPS = 1e-5

NSUB = 16
NPAD = 51200
STRIPE = NPAD // NSUB
FCHUNK = 256
ALL_CHUNKS = E // FCHUNK
FTRIPS = 195
C0_CHUNKS = 1563
HTRIPS = 97
NFPAD = 50048
FSTRIPE = NFPAD // NSUB
ZROWS = 184

ROWS = 1000
GRID = N // ROWS

_mesh = plsc.VectorSubcoreMesh(core_axis_name="c", subcore_axis_name="s")
_sc_params = pltpu.CompilerParams(use_tc_tiling_on_sc=False)


def _zero_fill_1d(buf, n):
    @pl.loop(0, n, step=16)
    def _(i):
        buf[pl.ds(i, 16)] = jnp.zeros((16,), jnp.float32)


def _zero_fill_2d(buf, rows, cols):
    @pl.loop(0, rows)
    def _(r):
        @pl.loop(0, cols, step=16)
        def _(c):
            buf[r, pl.ds(c, 16)] = jnp.zeros((16,), jnp.float32)


def _sc_degrees(ei_t):
    @functools.partial(
        pl.kernel,
        mesh=_mesh,
        compiler_params=_sc_params,
        out_type=(
            jax.ShapeDtypeStruct((2, NPAD), jnp.float32),
            jax.ShapeDtypeStruct((2, NPAD), jnp.float32),
        ),
        scratch_types=[
            pltpu.VMEM((4, 2, FCHUNK), jnp.int32),
            pltpu.VMEM((FCHUNK,), jnp.float32),
            pltpu.VMEM((STRIPE,), jnp.float32),
            pltpu.VMEM_SHARED((NPAD,), jnp.float32),
            pltpu.VMEM_SHARED((NPAD,), jnp.float32),
            pltpu.SemaphoreType.DMA,
            pltpu.SemaphoreType.DMA,
            pltpu.SemaphoreType.DMA,
            pltpu.SemaphoreType.DMA,
            pltpu.SemaphoreType.DMA,
            pltpu.SemaphoreType.DMA,
            pltpu.SemaphoreType.DMA,
            pltpu.SemaphoreType.DMA,
        ],
    )
    def k(ei, din_out, dout_out, ibuf, ones, zbuf, acc_in, acc_out,
          si0, si1, si2, si3, ss_i0, ss_i1, ss_o0, ss_o1):
        cid = lax.axis_index("c")
        sid = lax.axis_index("s")
        sems_i = (si0, si1, si2, si3)
        sems_si = (ss_i0, ss_i1)
        sems_so = (ss_o0, ss_o1)

        @pl.loop(0, FCHUNK // 16)
        def _(i):
            ones[pl.ds(i * 16, 16)] = jnp.ones((16,), jnp.float32)

        _zero_fill_1d(zbuf, STRIPE)
        pltpu.sync_copy(zbuf, acc_in.at[pl.ds(sid * STRIPE, STRIPE)])
        pltpu.sync_copy(zbuf, acc_out.at[pl.ds(sid * STRIPE, STRIPE)])
        plsc.subcore_barrier()


        trips = HTRIPS + (sid < (11 - cid)).astype(jnp.int32)

        def wait_slot(b):
            pltpu.make_async_copy(ones, acc_out.at[ibuf.at[b].at[0]],
                                  sems_so[b]).wait()
            pltpu.make_async_copy(ones, acc_in.at[ibuf.at[b].at[1]],
                                  sems_si[b]).wait()

        def issue(b, sb, db):
            pltpu.async_copy(ones, acc_out.at[sb], sems_so[b], add=True)
            pltpu.async_copy(ones, acc_in.at[db], sems_si[b], add=True)

        _edge_loop_strided(ei, ibuf, sems_i, cid * C0_CHUNKS + sid, NSUB,
                           trips, (HTRIPS + 4) // 4 + 1, wait_slot, issue)

        plsc.subcore_barrier()
        sl = pl.ds(sid * STRIPE, STRIPE)
        pltpu.sync_copy(acc_in.at[sl], din_out.at[cid].at[sl])
        pltpu.sync_copy(acc_out.at[sl], dout_out.at[cid].at[sl])

    return k(ei_t)


def _edge_loop_strided(ei_t, ibuf, sems_i, base, stride, trips, n_outer,
                       wait_slot, issue):
    for c0 in range(2):
        pltpu.async_copy(ei_t.at[base + stride * c0], ibuf.at[c0],
                         sems_i[c0])

    @pl.loop(0, n_outer)
    def _(g):
        for q in range(4):
            c = 4 * g + q

            @pl.when(c < trips)
            def _():
                @pl.when(c >= 2)
                def _():
                    wait_slot(q % 2)

                @pl.when(c + 2 < trips)
                def _():
                    pltpu.async_copy(
                        ei_t.at[base + stride * (c + 2)], ibuf.at[(q + 2) % 4],
                        sems_i[(q + 2) % 4])

                pltpu.make_async_copy(
                    ei_t.at[base + stride * c], ibuf.at[q], sems_i[q]).wait()
                issue(q % 2, ibuf.at[q].at[0], ibuf.at[q].at[1])

    for b in range(2):
        wait_slot(b)


def _sc_influence(ei_t, degf):
    @functools.partial(
        pl.kernel,
        mesh=_mesh,
        compiler_params=_sc_params,
        out_type=jax.ShapeDtypeStruct((2, NPAD), jnp.float32),
        scratch_types=[
            pltpu.VMEM((4, 2, FCHUNK), jnp.int32),
            pltpu.VMEM((2, FCHUNK), jnp.float32),
            pltpu.VMEM((STRIPE,), jnp.float32),
            pltpu.VMEM_SHARED((NPAD,), jnp.float32),
            pltpu.VMEM_SHARED((NPAD,), jnp.float32),
            pltpu.SemaphoreType.DMA,
            pltpu.SemaphoreType.DMA,
            pltpu.SemaphoreType.DMA,
            pltpu.SemaphoreType.DMA,
            pltpu.SemaphoreType.DMA,
            pltpu.SemaphoreType.DMA,
            pltpu.SemaphoreType.DMA,
        ],
    )
    def k(ei, dg, infl_out, ibuf, vals, zbuf, acc, dg_s,
          si0, si1, si2, si3, sem_g, ss0, ss1):
        cid = lax.axis_index("c")
        sid = lax.axis_index("s")
        sems_i = (si0, si1, si2, si3)
        sems_s = (ss0, ss1)

        _zero_fill_1d(zbuf, STRIPE)
        sl = pl.ds(sid * STRIPE, STRIPE)
        pltpu.sync_copy(zbuf, acc.at[sl])
        pltpu.sync_copy(dg.at[sl], dg_s.at[sl])
        plsc.subcore_barrier()

        trips = HTRIPS + (sid < (11 - cid)).astype(jnp.int32)

        def wait_slot(b):
            pltpu.make_async_copy(vals.at[b], acc.at[ibuf.at[b].at[0]],
                                  sems_s[b]).wait()

        def issue(b, sb, db):
            pltpu.async_copy(dg_s.at[db], vals.at[b], sem_g).wait()
            pltpu.async_copy(vals.at[b], acc.at[sb], sems_s[b], add=True)

        _edge_loop_strided(ei, ibuf, sems_i, cid * C0_CHUNKS + sid, NSUB,
                           trips, (HTRIPS + 4) // 4 + 1, wait_slot, issue)

        plsc.subcore_barrier()
        pltpu.sync_copy(acc.at[sl], infl_out.at[cid].at[sl])

    return k(ei_t, degf)


def _sc_scatter_features(ei_t, hws_a, hws_b):
    @functools.partial(
        pl.kernel,
        mesh=_mesh,
        compiler_params=_sc_params,
        out_type=jax.ShapeDtypeStruct((2, NFPAD, HD), jnp.float32),
        scratch_types=[
            pltpu.VMEM((4, 2, FCHUNK), jnp.int32),
            pltpu.VMEM((2, FCHUNK, HD), jnp.float32),
            pltpu.VMEM((ZROWS, HD), jnp.float32),
            pltpu.VMEM_SHARED((NFPAD, HD), jnp.float32),
            pltpu.SemaphoreType.DMA,
            pltpu.SemaphoreType.DMA,
            pltpu.SemaphoreType.DMA,
            pltpu.SemaphoreType.DMA,
            pltpu.SemaphoreType.DMA,
            pltpu.SemaphoreType.DMA,
            pltpu.SemaphoreType.DMA,
        ],
    )
    def k(ei, ha, hb, acc_out, ibuf, rows, zbuf, acc,
          si0, si1, si2, si3, sem_g, ss0, ss1):
        cid = lax.axis_index("c")
        sid = lax.axis_index("s")
        sems_i = (si0, si1, si2, si3)
        sems_s = (ss0, ss1)

        _zero_fill_2d(zbuf, ZROWS, HD)

        @pl.loop(0, FSTRIPE // ZROWS)
        def _(r):
            pltpu.sync_copy(zbuf, acc.at[pl.ds(sid * FSTRIPE + r * ZROWS, ZROWS)])

        plsc.subcore_barrier()

        trips = FTRIPS + (sid < 5).astype(jnp.int32)

        def wait_slot(b):
            pltpu.make_async_copy(rows.at[b], acc.at[ibuf.at[b].at[1]],
                                  sems_s[b]).wait()

        def edge_loop(table):
            def issue(b, sb, db):
                pltpu.async_copy(table.at[sb], rows.at[b], sem_g).wait()
                pltpu.async_copy(rows.at[b], acc.at[db], sems_s[b], add=True)

            _edge_loop_strided(ei, ibuf, sems_i, sid, NSUB, trips,
                               (FTRIPS + 4) // 4 + 1, wait_slot, issue)

        @pl.when(cid == 0)
        def _():
            edge_loop(ha)

        @pl.when(cid == 1)
        def _():
            edge_loop(hb)

        plsc.subcore_barrier()
        sl = pl.ds(sid * FSTRIPE, FSTRIPE)
        pltpu.sync_copy(acc.at[sl], acc_out.at[cid].at[sl])

    return k(ei_t, hws_a, hws_b)


_INV_BN = 1.0 / math.sqrt(1.0 + EPS)


def _tc_prep(din_parts, dout_parts):
    def body(din, dout, dinv_ref, degf_ref):
        deg = din[0:1, :] + din[1:2, :] + 1.0
        dinv_ref[...] = lax.rsqrt(deg)
        degf_ref[...] = dout[0:1, :] + dout[1:2, :]

    return pl.pallas_call(
        body,
        out_shape=(
            jax.ShapeDtypeStruct((1, NPAD), jnp.float32),
            jax.ShapeDtypeStruct((1, NPAD), jnp.float32),
        ),
    )(din_parts, dout_parts)


def _tc_structural(degf, infl_parts):
    def body(dg, ip, nd_ref, inf_ref):
        dout = dg[...]
        infl_sum = ip[0:1, :] + ip[1:2, :]
        maxd = jnp.max(dout)
        nd_ref[...] = jnp.where(maxd > 0, dout / jnp.maximum(maxd, 1e-12), dout)
        influence = jnp.where(dout > 0, infl_sum / jnp.maximum(dout, 1.0), 0.0)
        maxi = jnp.max(influence)
        inf_ref[...] = jnp.where(
            maxi > 0, influence / jnp.maximum(maxi, 1e-12), influence
        )

    return pl.pallas_call(
        body,
        out_shape=(
            jax.ShapeDtypeStruct((1, NPAD), jnp.float32),
            jax.ShapeDtypeStruct((1, NPAD), jnp.float32),
        ),
    )(degf, infl_parts)


def _split_out(p, out_ref):
    out_ref[0, :, :] = p[:, :HD]
    out_ref[1, :, :] = p[:, HD:]


def _tc_input_layer(x, W_in, b_in, W_g0, dinv_col):
    def body(x_ref, wi, bi, wg, dv, out_ref):
        h = jnp.dot(x_ref[...], wi[...], preferred_element_type=jnp.float32)
        h = h + bi[...]
        p = jnp.dot(h, wg[...], preferred_element_type=jnp.float32) * dv[...]
        _split_out(p, out_ref)

    return pl.pallas_call(
        body,
        grid=(GRID,),
        in_specs=[
            pl.BlockSpec((ROWS, D_IN), lambda i: (i, 0)),
            pl.BlockSpec((D_IN, DH), lambda i: (0, 0)),
            pl.BlockSpec((1, DH), lambda i: (0, 0)),
            pl.BlockSpec((DH, DH), lambda i: (0, 0)),
            pl.BlockSpec((ROWS, 1), lambda i: (i, 0)),
        ],
        out_specs=pl.BlockSpec((2, ROWS, HD), lambda i: (0, i, 0)),
        out_shape=jax.ShapeDtypeStruct((2, N, HD), jnp.float32),
    )(x, W_in, b_in, W_g0, dinv_col)


def _finish_layer(acc, hws, dv, b, g, be):
    m = jnp.concatenate([acc[0] + hws[0], acc[1] + hws[1]], axis=1)
    hn = dv * m + b
    return jnp.maximum((hn * _INV_BN) * g + be, 0.0)


def _tc_mid_layer(acc, hws, dinv_col, b, g, be, W_next):
    def body(acc_ref, hws_ref, dv_ref, b_ref, g_ref, be_ref, w_ref, out_ref):
        h = _finish_layer(
            acc_ref[...], hws_ref[...], dv_ref[...], b_ref[...], g_ref[...], be_ref[...]
        )
        p = jnp.dot(h, w_ref[...], preferred_element_type=jnp.float32) * dv_ref[...]
        _split_out(p, out_ref)

    return pl.pallas_call(
        body,
        grid=(GRID,),
        in_specs=[
            pl.BlockSpec((2, ROWS, HD), lambda i: (0, i, 0)),
            pl.BlockSpec((2, ROWS, HD), lambda i: (0, i, 0)),
            pl.BlockSpec((ROWS, 1), lambda i: (i, 0)),
            pl.BlockSpec((1, DH), lambda i: (0, 0)),
            pl.BlockSpec((1, DH), lambda i: (0, 0)),
            pl.BlockSpec((1, DH), lambda i: (0, 0)),
            pl.BlockSpec((DH, DH), lambda i: (0, 0)),
        ],
        out_specs=pl.BlockSpec((2, ROWS, HD), lambda i: (0, i, 0)),
        out_shape=jax.ShapeDtypeStruct((2, N, HD), jnp.float32),
    )(acc, hws, dinv_col, b, g, be, W_next)


def _tc_output(acc, hws, dinv_col, b, g, be, nd_col, inf_col, W_s1, b_s1, W_s2,
               b_s2, W_o1, b_o1, W_o2, b_o2, W_o3, b_o3):
    def body(acc_ref, hws_ref, dv_ref, b_ref, g_ref, be_ref, nd_ref, inf_ref,
             ws1, bs1, ws2, bs2, wo1, bo1, wo2, bo2, wo3, bo3, out_ref):
        h = _finish_layer(
            acc_ref[...], hws_ref[...], dv_ref[...], b_ref[...], g_ref[...], be_ref[...]
        )
        s_pre = nd_ref[...] * ws1[0:1, :] + inf_ref[...] * ws1[2:3, :] + bs1[...]
        se = jnp.dot(jnp.maximum(s_pre, 0.0), ws2[...],
                     preferred_element_type=jnp.float32) + bs2[...]
        hcat = jnp.concatenate([h, se], axis=1)
        o = jnp.dot(hcat, wo1[...], preferred_element_type=jnp.float32) + bo1[...]
        o = jnp.maximum(o, 0.0)
        o = jnp.dot(o, wo2[...], preferred_element_type=jnp.float32) + bo2[...]
        o = jnp.maximum(o, 0.0)
        o = jnp.dot(o, wo3[...], preferred_element_type=jnp.float32) + bo3[...]
        out_ref[...] = jax.nn.sigmoid(o)

    return pl.pallas_call(
        body,
        grid=(GRID,),
        in_specs=[
            pl.BlockSpec((2, ROWS, HD), lambda i: (0, i, 0)),
            pl.BlockSpec((2, ROWS, HD), lambda i: (0, i, 0)),
            pl.BlockSpec((ROWS, 1), lambda i: (i, 0)),
            pl.BlockSpec((1, DH), lambda i: (0, 0)),
            pl.BlockSpec((1, DH), lambda i: (0, 0)),
            pl.BlockSpec((1, DH), lambda i: (0, 0)),
            pl.BlockSpec((ROWS, 1), lambda i: (i, 0)),
            pl.BlockSpec((ROWS, 1), lambda i: (i, 0)),
            pl.BlockSpec((3, HD), lambda i: (0, 0)),
            pl.BlockSpec((1, HD), lambda i: (0, 0)),
            pl.BlockSpec((HD, DH), lambda i: (0, 0)),
            pl.BlockSpec((1, DH), lambda i: (0, 0)),
            pl.BlockSpec((2 * DH, DH), lambda i: (0, 0)),
            pl.BlockSpec((1, DH), lambda i: (0, 0)),
            pl.BlockSpec((DH, HD), lambda i: (0, 0)),
            pl.BlockSpec((1, HD), lambda i: (0, 0)),
            pl.BlockSpec((HD, 1), lambda i: (0, 0)),
            pl.BlockSpec((1, 1), lambda i: (0, 0)),
        ],
        out_specs=pl.BlockSpec((ROWS, 1), lambda i: (i, 0)),
        out_shape=jax.ShapeDtypeStruct((N, 1), jnp.float32),
    )(acc, hws, dinv_col, b, g, be, nd_col, inf_col, W_s1, b_s1, W_s2, b_s2,
      W_o1, b_o1, W_o2, b_o2, W_o3, b_o3)


def kernel(x, edge_index, W_in, b_in, W_g0, b_g0, W_g1, b_g1, W_g2, b_g2,
           bn_gamma, bn_beta, W_s1, b_s1, W_s2, b_s2, W_o1, b_o1, W_o2, b_o2,
           W_o3, b_o3):
    ei_t = edge_index.reshape(2, ALL_CHUNKS, FCHUNK).transpose(1, 0, 2)
    din_parts, dout_parts = _sc_degrees(ei_t)
    dinv_row, degf_row = _tc_prep(din_parts, dout_parts)
    degf = degf_row.reshape(NPAD)
    infl_parts = _sc_influence(ei_t, degf)
    nd_row, inf_row = _tc_structural(degf_row, infl_parts)

    dinv_col = dinv_row.reshape(NPAD, 1)[:N]
    nd_col = nd_row.reshape(NPAD, 1)[:N]
    inf_col = inf_row.reshape(NPAD, 1)[:N]

    b_in2 = b_in.reshape(1, DH)
    bs = (b_g0.reshape(1, DH), b_g1.reshape(1, DH), b_g2.reshape(1, DH))
    gs = tuple(bn_gamma[i].reshape(1, DH) for i in range(3))
    bes = tuple(bn_beta[i].reshape(1, DH) for i in range(3))

    hws = _tc_input_layer(x, W_in, b_in2, W_g0, dinv_col)
    acc = _sc_scatter_features(ei_t, hws[0], hws[1])
    hws1 = _tc_mid_layer(acc, hws, dinv_col, bs[0], gs[0], bes[0], W_g1)
    acc1 = _sc_scatter_features(ei_t, hws1[0], hws1[1])
    hws2 = _tc_mid_layer(acc1, hws1, dinv_col, bs[1], gs[1], bes[1], W_g2)
    acc2 = _sc_scatter_features(ei_t, hws2[0], hws2[1])

    return _tc_output(
        acc2, hws2, dinv_col, bs[2], gs[2], bes[2], nd_col, inf_col,
        W_s1, b_s1.reshape(1, HD), W_s2, b_s2.reshape(1, DH),
        W_o1, b_o1.reshape(1, DH), W_o2, b_o2.reshape(1, HD),
        W_o3, b_o3.reshape(1, 1),
    )


# speedup vs baseline: 31.7091x; 1.1487x over previous
import functools
import math

import jax
import jax.numpy as jnp
from jax import lax
from jax.experimental import pallas as pl
from jax.experimental.pallas import tpu as pltpu
from jax.experimental.pallas import tpu_sc as plsc

N = 50000
E = 800000
D_IN = 128
DH = 64
HD = DH // 2
EPS = 1e-5

NSUB = 16
NPAD = 51200
STRIPE = NPAD // NSUB
FCHUNK = 256
ALL_CHUNKS = E // FCHUNK
FTRIPS = 195
C0_CHUNKS = 1563
HTRIPS = 97
NFPAD = 50048
FSTRIPE = NFPAD // NSUB
ZROWS = 184

ROWS = 2000
GRID = N // ROWS

_mesh = plsc.VectorSubcoreMesh(core_axis_name="c", subcore_axis_name="s")
_sc_params = pltpu.CompilerParams(use_tc_tiling_on_sc=False)


def _zero_fill_1d(buf, n):
    @pl.loop(0, n, step=16)
    def _(i):
        buf[pl.ds(i, 16)] = jnp.zeros((16,), jnp.float32)


def _zero_fill_2d(buf, rows, cols):
    @pl.loop(0, rows)
    def _(r):
        @pl.loop(0, cols, step=16)
        def _(c):
            buf[r, pl.ds(c, 16)] = jnp.zeros((16,), jnp.float32)


def _sc_degrees(ei_t):
    @functools.partial(
        pl.kernel,
        mesh=_mesh,
        compiler_params=_sc_params,
        out_type=(
            jax.ShapeDtypeStruct((2, NPAD), jnp.float32),
            jax.ShapeDtypeStruct((2, NPAD), jnp.float32),
        ),
        scratch_types=[
            pltpu.VMEM((4, 2, FCHUNK), jnp.int32),
            pltpu.VMEM((FCHUNK,), jnp.float32),
            pltpu.VMEM((STRIPE,), jnp.float32),
            pltpu.VMEM_SHARED((NPAD,), jnp.float32),
            pltpu.VMEM_SHARED((NPAD,), jnp.float32),
            pltpu.SemaphoreType.DMA,
            pltpu.SemaphoreType.DMA,
            pltpu.SemaphoreType.DMA,
            pltpu.SemaphoreType.DMA,
            pltpu.SemaphoreType.DMA,
            pltpu.SemaphoreType.DMA,
            pltpu.SemaphoreType.DMA,
            pltpu.SemaphoreType.DMA,
        ],
    )
    def k(ei, din_out, dout_out, ibuf, ones, zbuf, acc_in, acc_out,
          si0, si1, si2, si3, ss_i0, ss_i1, ss_o0, ss_o1):
        cid = lax.axis_index("c")
        sid = lax.axis_index("s")
        sems_i = (si0, si1, si2, si3)
        sems_si = (ss_i0, ss_i1)
        sems_so = (ss_o0, ss_o1)

        @pl.loop(0, FCHUNK // 16)
        def _(i):
            ones[pl.ds(i * 16, 16)] = jnp.ones((16,), jnp.float32)

        _zero_fill_1d(zbuf, STRIPE)
        pltpu.sync_copy(zbuf, acc_in.at[pl.ds(sid * STRIPE, STRIPE)])
        pltpu.sync_copy(zbuf, acc_out.at[pl.ds(sid * STRIPE, STRIPE)])
        plsc.subcore_barrier()


        trips = HTRIPS + (sid < (11 - cid)).astype(jnp.int32)

        def wait_slot(b):
            pltpu.make_async_copy(ones, acc_out.at[ibuf.at[b].at[0]],
                                  sems_so[b]).wait()
            pltpu.make_async_copy(ones, acc_in.at[ibuf.at[b].at[1]],
                                  sems_si[b]).wait()

        def issue(b, sb, db):
            pltpu.async_copy(ones, acc_out.at[sb], sems_so[b], add=True)
            pltpu.async_copy(ones, acc_in.at[db], sems_si[b], add=True)

        _edge_loop_strided(ei, ibuf, sems_i, cid * C0_CHUNKS + sid, NSUB,
                           trips, (HTRIPS + 4) // 4 + 1, wait_slot, issue)

        plsc.subcore_barrier()
        sl = pl.ds(sid * STRIPE, STRIPE)
        pltpu.sync_copy(acc_in.at[sl], din_out.at[cid].at[sl])
        pltpu.sync_copy(acc_out.at[sl], dout_out.at[cid].at[sl])

    return k(ei_t)


def _edge_loop_strided(ei_t, ibuf, sems_i, base, stride, trips, n_outer,
                       wait_slot, issue):
    for c0 in range(2):
        pltpu.async_copy(ei_t.at[base + stride * c0], ibuf.at[c0],
                         sems_i[c0])

    @pl.loop(0, n_outer)
    def _(g):
        for q in range(4):
            c = 4 * g + q

            @pl.when(c < trips)
            def _():
                @pl.when(c >= 2)
                def _():
                    wait_slot(q % 2)

                @pl.when(c + 2 < trips)
                def _():
                    pltpu.async_copy(
                        ei_t.at[base + stride * (c + 2)], ibuf.at[(q + 2) % 4],
                        sems_i[(q + 2) % 4])

                pltpu.make_async_copy(
                    ei_t.at[base + stride * c], ibuf.at[q], sems_i[q]).wait()
                issue(q % 2, ibuf.at[q].at[0], ibuf.at[q].at[1])

    for b in range(2):
        wait_slot(b)


def _sc_influence(ei_t, degf):
    @functools.partial(
        pl.kernel,
        mesh=_mesh,
        compiler_params=_sc_params,
        out_type=jax.ShapeDtypeStruct((2, NPAD), jnp.float32),
        scratch_types=[
            pltpu.VMEM((4, 2, FCHUNK), jnp.int32),
            pltpu.VMEM((2, FCHUNK), jnp.float32),
            pltpu.VMEM((STRIPE,), jnp.float32),
            pltpu.VMEM_SHARED((NPAD,), jnp.float32),
            pltpu.VMEM_SHARED((NPAD,), jnp.float32),
            pltpu.SemaphoreType.DMA,
            pltpu.SemaphoreType.DMA,
            pltpu.SemaphoreType.DMA,
            pltpu.SemaphoreType.DMA,
            pltpu.SemaphoreType.DMA,
            pltpu.SemaphoreType.DMA,
            pltpu.SemaphoreType.DMA,
        ],
    )
    def k(ei, dg, infl_out, ibuf, vals, zbuf, acc, dg_s,
          si0, si1, si2, si3, sem_g, ss0, ss1):
        cid = lax.axis_index("c")
        sid = lax.axis_index("s")
        sems_i = (si0, si1, si2, si3)
        sems_s = (ss0, ss1)

        _zero_fill_1d(zbuf, STRIPE)
        sl = pl.ds(sid * STRIPE, STRIPE)
        pltpu.sync_copy(zbuf, acc.at[sl])
        pltpu.sync_copy(dg.at[sl], dg_s.at[sl])
        plsc.subcore_barrier()

        trips = HTRIPS + (sid < (11 - cid)).astype(jnp.int32)

        def wait_slot(b):
            pltpu.make_async_copy(vals.at[b], acc.at[ibuf.at[b].at[0]],
                                  sems_s[b]).wait()

        def issue(b, sb, db):
            pltpu.async_copy(dg_s.at[db], vals.at[b], sem_g).wait()
            pltpu.async_copy(vals.at[b], acc.at[sb], sems_s[b], add=True)

        _edge_loop_strided(ei, ibuf, sems_i, cid * C0_CHUNKS + sid, NSUB,
                           trips, (HTRIPS + 4) // 4 + 1, wait_slot, issue)

        plsc.subcore_barrier()
        pltpu.sync_copy(acc.at[sl], infl_out.at[cid].at[sl])

    return k(ei_t, degf)


def _sc_scatter_features(ei_t, hws):
    @functools.partial(
        pl.kernel,
        mesh=_mesh,
        compiler_params=_sc_params,
        out_type=jax.ShapeDtypeStruct((2, NFPAD, HD), jnp.float32),
        scratch_types=[
            pltpu.VMEM((4, 2, FCHUNK), jnp.int32),
            pltpu.VMEM((2, FCHUNK, HD), jnp.float32),
            pltpu.VMEM((ZROWS, HD), jnp.float32),
            pltpu.VMEM_SHARED((NFPAD, HD), jnp.float32),
            pltpu.SemaphoreType.DMA,
            pltpu.SemaphoreType.DMA,
            pltpu.SemaphoreType.DMA,
            pltpu.SemaphoreType.DMA,
            pltpu.SemaphoreType.DMA,
            pltpu.SemaphoreType.DMA,
            pltpu.SemaphoreType.DMA,
        ],
    )
    def k(ei, hws_ref, acc_out, ibuf, rows, zbuf, acc,
          si0, si1, si2, si3, sem_g, ss0, ss1):
        cid = lax.axis_index("c")
        sid = lax.axis_index("s")
        sems_i = (si0, si1, si2, si3)
        sems_s = (ss0, ss1)

        _zero_fill_2d(zbuf, ZROWS, HD)

        @pl.loop(0, FSTRIPE // ZROWS)
        def _(r):
            pltpu.sync_copy(zbuf, acc.at[pl.ds(sid * FSTRIPE + r * ZROWS, ZROWS)])

        plsc.subcore_barrier()

        trips = FTRIPS + (sid < 5).astype(jnp.int32)

        def wait_slot(b):
            pltpu.make_async_copy(rows.at[b], acc.at[ibuf.at[b].at[1]],
                                  sems_s[b]).wait()

        def edge_loop(table):
            def issue(b, sb, db):
                pltpu.async_copy(table.at[sb], rows.at[b], sem_g).wait()
                pltpu.async_copy(rows.at[b], acc.at[db], sems_s[b], add=True)

            _edge_loop_strided(ei, ibuf, sems_i, sid, NSUB, trips,
                               (FTRIPS + 4) // 4 + 1, wait_slot, issue)

        @pl.when(cid == 0)
        def _():
            edge_loop(hws_ref.at[0])

        @pl.when(cid == 1)
        def _():
            edge_loop(hws_ref.at[1])

        plsc.subcore_barrier()
        sl = pl.ds(sid * FSTRIPE, FSTRIPE)
        pltpu.sync_copy(acc.at[sl], acc_out.at[cid].at[sl])

    return k(ei_t, hws)


_INV_BN = 1.0 / math.sqrt(1.0 + EPS)


def _tc_prep(din_parts, dout_parts):
    def body(din, dout, dinv_ref, degf_ref):
        deg = din[0:1, :] + din[1:2, :] + 1.0
        dinv_ref[...] = lax.rsqrt(deg)
        degf_ref[...] = dout[0:1, :] + dout[1:2, :]

    return pl.pallas_call(
        body,
        out_shape=(
            jax.ShapeDtypeStruct((1, NPAD), jnp.float32),
            jax.ShapeDtypeStruct((1, NPAD), jnp.float32),
        ),
    )(din_parts, dout_parts)


def _tc_structural(degf, infl_parts):
    def body(dg, ip, nd_ref, inf_ref):
        dout = dg[...]
        infl_sum = ip[0:1, :] + ip[1:2, :]
        maxd = jnp.max(dout)
        nd_ref[...] = jnp.where(maxd > 0, dout / jnp.maximum(maxd, 1e-12), dout)
        influence = jnp.where(dout > 0, infl_sum / jnp.maximum(dout, 1.0), 0.0)
        maxi = jnp.max(influence)
        inf_ref[...] = jnp.where(
            maxi > 0, influence / jnp.maximum(maxi, 1e-12), influence
        )

    return pl.pallas_call(
        body,
        out_shape=(
            jax.ShapeDtypeStruct((1, NPAD), jnp.float32),
            jax.ShapeDtypeStruct((1, NPAD), jnp.float32),
        ),
    )(degf, infl_parts)


def _split_out(p, out_ref):
    out_ref[0, :, :] = p[:, :HD]
    out_ref[1, :, :] = p[:, HD:]


def _tc_input_layer(x, W_in, b_in, W_g0, dinv_col):
    def body(x_ref, wi, bi, wg, dv, out_ref):
        h = jnp.dot(x_ref[...], wi[...], preferred_element_type=jnp.float32)
        h = h + bi[...]
        p = jnp.dot(h, wg[...], preferred_element_type=jnp.float32) * dv[...]
        _split_out(p, out_ref)

    return pl.pallas_call(
        body,
        grid=(GRID,),
        in_specs=[
            pl.BlockSpec((ROWS, D_IN), lambda i: (i, 0)),
            pl.BlockSpec((D_IN, DH), lambda i: (0, 0)),
            pl.BlockSpec((1, DH), lambda i: (0, 0)),
            pl.BlockSpec((DH, DH), lambda i: (0, 0)),
            pl.BlockSpec((ROWS, 1), lambda i: (i, 0)),
        ],
        out_specs=pl.BlockSpec((2, ROWS, HD), lambda i: (0, i, 0)),
        out_shape=jax.ShapeDtypeStruct((2, N, HD), jnp.float32),
    )(x, W_in, b_in, W_g0, dinv_col)


def _finish_layer(acc, hws, dv, b, g, be):
    m = jnp.concatenate([acc[0] + hws[0], acc[1] + hws[1]], axis=1)
    hn = dv * m + b
    return jnp.maximum((hn * _INV_BN) * g + be, 0.0)


def _tc_mid_layer(acc, hws, dinv_col, b, g, be, W_next):
    def body(acc_ref, hws_ref, dv_ref, b_ref, g_ref, be_ref, w_ref, out_ref):
        h = _finish_layer(
            acc_ref[...], hws_ref[...], dv_ref[...], b_ref[...], g_ref[...], be_ref[...]
        )
        p = jnp.dot(h, w_ref[...], preferred_element_type=jnp.float32) * dv_ref[...]
        _split_out(p, out_ref)

    return pl.pallas_call(
        body,
        grid=(GRID,),
        in_specs=[
            pl.BlockSpec((2, ROWS, HD), lambda i: (0, i, 0)),
            pl.BlockSpec((2, ROWS, HD), lambda i: (0, i, 0)),
            pl.BlockSpec((ROWS, 1), lambda i: (i, 0)),
            pl.BlockSpec((1, DH), lambda i: (0, 0)),
            pl.BlockSpec((1, DH), lambda i: (0, 0)),
            pl.BlockSpec((1, DH), lambda i: (0, 0)),
            pl.BlockSpec((DH, DH), lambda i: (0, 0)),
        ],
        out_specs=pl.BlockSpec((2, ROWS, HD), lambda i: (0, i, 0)),
        out_shape=jax.ShapeDtypeStruct((2, N, HD), jnp.float32),
    )(acc, hws, dinv_col, b, g, be, W_next)


def _tc_output(acc, hws, dinv_col, b, g, be, nd_col, inf_col, W_s1, b_s1, W_s2,
               b_s2, W_o1, b_o1, W_o2, b_o2, W_o3, b_o3):
    def body(acc_ref, hws_ref, dv_ref, b_ref, g_ref, be_ref, nd_ref, inf_ref,
             ws1, bs1, ws2, bs2, wo1, bo1, wo2, bo2, wo3, bo3, out_ref):
        h = _finish_layer(
            acc_ref[...], hws_ref[...], dv_ref[...], b_ref[...], g_ref[...], be_ref[...]
        )
        s_pre = nd_ref[...] * ws1[0:1, :] + inf_ref[...] * ws1[2:3, :] + bs1[...]
        se = jnp.dot(jnp.maximum(s_pre, 0.0), ws2[...],
                     preferred_element_type=jnp.float32) + bs2[...]
        hcat = jnp.concatenate([h, se], axis=1)
        o = jnp.dot(hcat, wo1[...], preferred_element_type=jnp.float32) + bo1[...]
        o = jnp.maximum(o, 0.0)
        o = jnp.dot(o, wo2[...], preferred_element_type=jnp.float32) + bo2[...]
        o = jnp.maximum(o, 0.0)
        o = jnp.dot(o, wo3[...], preferred_element_type=jnp.float32) + bo3[...]
        out_ref[...] = jax.nn.sigmoid(o)

    return pl.pallas_call(
        body,
        grid=(GRID,),
        in_specs=[
            pl.BlockSpec((2, ROWS, HD), lambda i: (0, i, 0)),
            pl.BlockSpec((2, ROWS, HD), lambda i: (0, i, 0)),
            pl.BlockSpec((ROWS, 1), lambda i: (i, 0)),
            pl.BlockSpec((1, DH), lambda i: (0, 0)),
            pl.BlockSpec((1, DH), lambda i: (0, 0)),
            pl.BlockSpec((1, DH), lambda i: (0, 0)),
            pl.BlockSpec((ROWS, 1), lambda i: (i, 0)),
            pl.BlockSpec((ROWS, 1), lambda i: (i, 0)),
            pl.BlockSpec((3, HD), lambda i: (0, 0)),
            pl.BlockSpec((1, HD), lambda i: (0, 0)),
            pl.BlockSpec((HD, DH), lambda i: (0, 0)),
            pl.BlockSpec((1, DH), lambda i: (0, 0)),
            pl.BlockSpec((2 * DH, DH), lambda i: (0, 0)),
            pl.BlockSpec((1, DH), lambda i: (0, 0)),
            pl.BlockSpec((DH, HD), lambda i: (0, 0)),
            pl.BlockSpec((1, HD), lambda i: (0, 0)),
            pl.BlockSpec((HD, 1), lambda i: (0, 0)),
            pl.BlockSpec((1, 1), lambda i: (0, 0)),
        ],
        out_specs=pl.BlockSpec((ROWS, 1), lambda i: (i, 0)),
        out_shape=jax.ShapeDtypeStruct((N, 1), jnp.float32),
    )(acc, hws, dinv_col, b, g, be, nd_col, inf_col, W_s1, b_s1, W_s2, b_s2,
      W_o1, b_o1, W_o2, b_o2, W_o3, b_o3)


def kernel(x, edge_index, W_in, b_in, W_g0, b_g0, W_g1, b_g1, W_g2, b_g2,
           bn_gamma, bn_beta, W_s1, b_s1, W_s2, b_s2, W_o1, b_o1, W_o2, b_o2,
           W_o3, b_o3):
    ei_t = edge_index.reshape(2, ALL_CHUNKS, FCHUNK).transpose(1, 0, 2)
    din_parts, dout_parts = _sc_degrees(ei_t)
    dinv_row, degf_row = _tc_prep(din_parts, dout_parts)
    degf = degf_row.reshape(NPAD)
    infl_parts = _sc_influence(ei_t, degf)
    nd_row, inf_row = _tc_structural(degf_row, infl_parts)

    dinv_col = dinv_row.reshape(NPAD, 1)[:N]
    nd_col = nd_row.reshape(NPAD, 1)[:N]
    inf_col = inf_row.reshape(NPAD, 1)[:N]

    b_in2 = b_in.reshape(1, DH)
    bs = (b_g0.reshape(1, DH), b_g1.reshape(1, DH), b_g2.reshape(1, DH))
    gs = tuple(bn_gamma[i].reshape(1, DH) for i in range(3))
    bes = tuple(bn_beta[i].reshape(1, DH) for i in range(3))

    hws = _tc_input_layer(x, W_in, b_in2, W_g0, dinv_col)
    acc = _sc_scatter_features(ei_t, hws)
    hws1 = _tc_mid_layer(acc, hws, dinv_col, bs[0], gs[0], bes[0], W_g1)
    acc1 = _sc_scatter_features(ei_t, hws1)
    hws2 = _tc_mid_layer(acc1, hws1, dinv_col, bs[1], gs[1], bes[1], W_g2)
    acc2 = _sc_scatter_features(ei_t, hws2)

    return _tc_output(
        acc2, hws2, dinv_col, bs[2], gs[2], bes[2], nd_col, inf_col,
        W_s1, b_s1.reshape(1, HD), W_s2, b_s2.reshape(1, DH),
        W_o1, b_o1.reshape(1, DH), W_o2, b_o2.reshape(1, HD),
        W_o3, b_o3.reshape(1, 1),
    )


# speedup vs baseline: 40.0515x; 1.2631x over previous
import functools
import math

import jax
import jax.numpy as jnp
from jax import lax
from jax.experimental import pallas as pl
from jax.experimental.pallas import tpu as pltpu
from jax.experimental.pallas import tpu_sc as plsc

N = 50000
E = 800000
D_IN = 128
DH = 64
HD = DH // 2
EPS = 1e-5

NSUB = 16
NPAD = 51200
STRIPE = NPAD // NSUB
FCHUNK = 256
ALL_CHUNKS = E // FCHUNK
FTRIPS = 195
C0_CHUNKS = 1563
HTRIPS = 97
NFPAD = 50048
FSTRIPE = NFPAD // NSUB
ZROWS = 184

ROWS = 2176
PACK = ROWS // 4
NPACK = N * HD // 128
APACK = NFPAD * HD // 128
GRID = -(-N // ROWS)

_mesh = plsc.VectorSubcoreMesh(core_axis_name="c", subcore_axis_name="s")
_sc_params = pltpu.CompilerParams(use_tc_tiling_on_sc=False)


def _zero_fill_1d(buf, n):
    @pl.loop(0, n, step=16)
    def _(i):
        buf[pl.ds(i, 16)] = jnp.zeros((16,), jnp.float32)


def _zero_fill_2d(buf, rows, cols):
    @pl.loop(0, rows)
    def _(r):
        @pl.loop(0, cols, step=16)
        def _(c):
            buf[r, pl.ds(c, 16)] = jnp.zeros((16,), jnp.float32)


def _sc_degrees(ei_t):
    @functools.partial(
        pl.kernel,
        mesh=_mesh,
        compiler_params=_sc_params,
        out_type=(
            jax.ShapeDtypeStruct((2, NPAD), jnp.float32),
            jax.ShapeDtypeStruct((2, NPAD), jnp.float32),
        ),
        scratch_types=[
            pltpu.VMEM((4, 2, FCHUNK), jnp.int32),
            pltpu.VMEM((FCHUNK,), jnp.float32),
            pltpu.VMEM((STRIPE,), jnp.float32),
            pltpu.VMEM_SHARED((NPAD,), jnp.float32),
            pltpu.VMEM_SHARED((NPAD,), jnp.float32),
            pltpu.SemaphoreType.DMA,
            pltpu.SemaphoreType.DMA,
            pltpu.SemaphoreType.DMA,
            pltpu.SemaphoreType.DMA,
            pltpu.SemaphoreType.DMA,
            pltpu.SemaphoreType.DMA,
            pltpu.SemaphoreType.DMA,
            pltpu.SemaphoreType.DMA,
        ],
    )
    def k(ei, din_out, dout_out, ibuf, ones, zbuf, acc_in, acc_out,
          si0, si1, si2, si3, ss_i0, ss_i1, ss_o0, ss_o1):
        cid = lax.axis_index("c")
        sid = lax.axis_index("s")
        sems_i = (si0, si1, si2, si3)
        sems_si = (ss_i0, ss_i1)
        sems_so = (ss_o0, ss_o1)

        @pl.loop(0, FCHUNK // 16)
        def _(i):
            ones[pl.ds(i * 16, 16)] = jnp.ones((16,), jnp.float32)

        _zero_fill_1d(zbuf, STRIPE)
        pltpu.sync_copy(zbuf, acc_in.at[pl.ds(sid * STRIPE, STRIPE)])
        pltpu.sync_copy(zbuf, acc_out.at[pl.ds(sid * STRIPE, STRIPE)])
        plsc.subcore_barrier()


        trips = HTRIPS + (sid < (11 - cid)).astype(jnp.int32)

        def wait_slot(b):
            pltpu.make_async_copy(ones, acc_out.at[ibuf.at[b].at[0]],
                                  sems_so[b]).wait()
            pltpu.make_async_copy(ones, acc_in.at[ibuf.at[b].at[1]],
                                  sems_si[b]).wait()

        def issue(b, sb, db):
            pltpu.async_copy(ones, acc_out.at[sb], sems_so[b], add=True)
            pltpu.async_copy(ones, acc_in.at[db], sems_si[b], add=True)

        _edge_loop_strided(ei, ibuf, sems_i, cid * C0_CHUNKS + sid, NSUB,
                           trips, (HTRIPS + 4) // 4 + 1, wait_slot, issue)

        plsc.subcore_barrier()
        sl = pl.ds(sid * STRIPE, STRIPE)
        pltpu.sync_copy(acc_in.at[sl], din_out.at[cid].at[sl])
        pltpu.sync_copy(acc_out.at[sl], dout_out.at[cid].at[sl])

    return k(ei_t)


def _edge_loop_strided(ei_t, ibuf, sems_i, base, stride, trips, n_outer,
                       wait_slot, issue):
    for c0 in range(2):
        pltpu.async_copy(ei_t.at[base + stride * c0], ibuf.at[c0],
                         sems_i[c0])

    @pl.loop(0, n_outer)
    def _(g):
        for q in range(4):
            c = 4 * g + q

            @pl.when(c < trips)
            def _():
                @pl.when(c >= 2)
                def _():
                    wait_slot(q % 2)

                @pl.when(c + 2 < trips)
                def _():
                    pltpu.async_copy(
                        ei_t.at[base + stride * (c + 2)], ibuf.at[(q + 2) % 4],
                        sems_i[(q + 2) % 4])

                pltpu.make_async_copy(
                    ei_t.at[base + stride * c], ibuf.at[q], sems_i[q]).wait()
                issue(q % 2, ibuf.at[q].at[0], ibuf.at[q].at[1])

    for b in range(2):
        wait_slot(b)


def _sc_influence(ei_t, degf):
    @functools.partial(
        pl.kernel,
        mesh=_mesh,
        compiler_params=_sc_params,
        out_type=jax.ShapeDtypeStruct((2, NPAD), jnp.float32),
        scratch_types=[
            pltpu.VMEM((4, 2, FCHUNK), jnp.int32),
            pltpu.VMEM((2, FCHUNK), jnp.float32),
            pltpu.VMEM((STRIPE,), jnp.float32),
            pltpu.VMEM_SHARED((NPAD,), jnp.float32),
            pltpu.VMEM_SHARED((NPAD,), jnp.float32),
            pltpu.SemaphoreType.DMA,
            pltpu.SemaphoreType.DMA,
            pltpu.SemaphoreType.DMA,
            pltpu.SemaphoreType.DMA,
            pltpu.SemaphoreType.DMA,
            pltpu.SemaphoreType.DMA,
            pltpu.SemaphoreType.DMA,
        ],
    )
    def k(ei, dg, infl_out, ibuf, vals, zbuf, acc, dg_s,
          si0, si1, si2, si3, sem_g, ss0, ss1):
        cid = lax.axis_index("c")
        sid = lax.axis_index("s")
        sems_i = (si0, si1, si2, si3)
        sems_s = (ss0, ss1)

        _zero_fill_1d(zbuf, STRIPE)
        sl = pl.ds(sid * STRIPE, STRIPE)
        pltpu.sync_copy(zbuf, acc.at[sl])
        pltpu.sync_copy(dg.at[sl], dg_s.at[sl])
        plsc.subcore_barrier()

        trips = HTRIPS + (sid < (11 - cid)).astype(jnp.int32)

        def wait_slot(b):
            pltpu.make_async_copy(vals.at[b], acc.at[ibuf.at[b].at[0]],
                                  sems_s[b]).wait()

        def issue(b, sb, db):
            pltpu.async_copy(dg_s.at[db], vals.at[b], sem_g).wait()
            pltpu.async_copy(vals.at[b], acc.at[sb], sems_s[b], add=True)

        _edge_loop_strided(ei, ibuf, sems_i, cid * C0_CHUNKS + sid, NSUB,
                           trips, (HTRIPS + 4) // 4 + 1, wait_slot, issue)

        plsc.subcore_barrier()
        pltpu.sync_copy(acc.at[sl], infl_out.at[cid].at[sl])

    return k(ei_t, degf)


def _sc_scatter_features(ei_t, hws):
    @functools.partial(
        pl.kernel,
        mesh=_mesh,
        compiler_params=_sc_params,
        out_type=jax.ShapeDtypeStruct((2, NFPAD, HD), jnp.float32),
        scratch_types=[
            pltpu.VMEM((4, 2, FCHUNK), jnp.int32),
            pltpu.VMEM((2, FCHUNK, HD), jnp.float32),
            pltpu.VMEM((ZROWS, HD), jnp.float32),
            pltpu.VMEM_SHARED((NFPAD, HD), jnp.float32),
            pltpu.SemaphoreType.DMA,
            pltpu.SemaphoreType.DMA,
            pltpu.SemaphoreType.DMA,
            pltpu.SemaphoreType.DMA,
            pltpu.SemaphoreType.DMA,
            pltpu.SemaphoreType.DMA,
            pltpu.SemaphoreType.DMA,
        ],
    )
    def k(ei, hws_ref, acc_out, ibuf, rows, zbuf, acc,
          si0, si1, si2, si3, sem_g, ss0, ss1):
        cid = lax.axis_index("c")
        sid = lax.axis_index("s")
        sems_i = (si0, si1, si2, si3)
        sems_s = (ss0, ss1)

        _zero_fill_2d(zbuf, ZROWS, HD)

        @pl.loop(0, FSTRIPE // ZROWS)
        def _(r):
            pltpu.sync_copy(zbuf, acc.at[pl.ds(sid * FSTRIPE + r * ZROWS, ZROWS)])

        plsc.subcore_barrier()

        trips = FTRIPS + (sid < 5).astype(jnp.int32)

        def wait_slot(b):
            pltpu.make_async_copy(rows.at[b], acc.at[ibuf.at[b].at[1]],
                                  sems_s[b]).wait()

        def edge_loop(table):
            def issue(b, sb, db):
                pltpu.async_copy(table.at[sb], rows.at[b], sem_g).wait()
                pltpu.async_copy(rows.at[b], acc.at[db], sems_s[b], add=True)

            _edge_loop_strided(ei, ibuf, sems_i, sid, NSUB, trips,
                               (FTRIPS + 4) // 4 + 1, wait_slot, issue)

        @pl.when(cid == 0)
        def _():
            edge_loop(hws_ref.at[0])

        @pl.when(cid == 1)
        def _():
            edge_loop(hws_ref.at[1])

        plsc.subcore_barrier()
        sl = pl.ds(sid * FSTRIPE, FSTRIPE)
        pltpu.sync_copy(acc.at[sl], acc_out.at[cid].at[sl])

    return k(ei_t, hws)


_INV_BN = 1.0 / math.sqrt(1.0 + EPS)


def _tc_prep(din_parts, dout_parts):
    def body(din, dout, dinv_ref, degf_ref):
        deg = din[0:1, :] + din[1:2, :] + 1.0
        dinv_ref[...] = lax.rsqrt(deg)
        degf_ref[...] = dout[0:1, :] + dout[1:2, :]

    return pl.pallas_call(
        body,
        out_shape=(
            jax.ShapeDtypeStruct((1, NPAD), jnp.float32),
            jax.ShapeDtypeStruct((1, NPAD), jnp.float32),
        ),
    )(din_parts, dout_parts)


def _tc_structural(degf, infl_parts):
    def body(dg, ip, nd_ref, inf_ref):
        dout = dg[...]
        infl_sum = ip[0:1, :] + ip[1:2, :]
        maxd = jnp.max(dout)
        nd_ref[...] = jnp.where(maxd > 0, dout / jnp.maximum(maxd, 1e-12), dout)
        influence = jnp.where(dout > 0, infl_sum / jnp.maximum(dout, 1.0), 0.0)
        maxi = jnp.max(influence)
        inf_ref[...] = jnp.where(
            maxi > 0, influence / jnp.maximum(maxi, 1e-12), influence
        )

    return pl.pallas_call(
        body,
        out_shape=(
            jax.ShapeDtypeStruct((1, NPAD), jnp.float32),
            jax.ShapeDtypeStruct((1, NPAD), jnp.float32),
        ),
    )(degf, infl_parts)


def _split_out(p, out_ref):
    out_ref[0, :, :] = p[:, :HD]
    out_ref[1, :, :] = p[:, HD:]


def _tc_input_layer(x, W_in, b_in, W_g0, dinv_col):
    def body(x_ref, wi, bi, wg, dv, out_ref):
        h = jnp.dot(x_ref[...], wi[...], preferred_element_type=jnp.float32)
        h = h + bi[...]
        p = jnp.dot(h, wg[...], preferred_element_type=jnp.float32) * dv[...]
        _split_out(p, out_ref)

    return pl.pallas_call(
        body,
        grid=(GRID,),
        in_specs=[
            pl.BlockSpec((ROWS, D_IN), lambda i: (i, 0)),
            pl.BlockSpec((D_IN, DH), lambda i: (0, 0)),
            pl.BlockSpec((1, DH), lambda i: (0, 0)),
            pl.BlockSpec((DH, DH), lambda i: (0, 0)),
            pl.BlockSpec((ROWS, 1), lambda i: (i, 0)),
        ],
        out_specs=pl.BlockSpec((2, ROWS, HD), lambda i: (0, i, 0)),
        out_shape=jax.ShapeDtypeStruct((2, N, HD), jnp.float32),
    )(x, W_in, b_in, W_g0, dinv_col)


def _packed_h(acc_ref, hws_ref, dv, bA, bB, gA, gB, beA, beB):
    a = acc_ref[0, :, :] + hws_ref[0, :, :]
    b_ = acc_ref[1, :, :] + hws_ref[1, :, :]
    hA = jnp.maximum((dv * a + bA) * _INV_BN * gA + beA, 0.0)
    hB = jnp.maximum((dv * b_ + bB) * _INV_BN * gB + beB, 0.0)
    return hA, hB


def _tc_mid_layer(acc, hws, dinv_p, bA, bB, gA, gB, beA, beB,
                  DAA, DBA, DAB, DBB):
    def body(acc_ref, hws_ref, dv_ref, bA_r, bB_r, gA_r, gB_r, beA_r, beB_r,
             daa, dba, dab, dbb, out_ref):
        hA, hB = _packed_h(acc_ref, hws_ref, dv_ref[...], bA_r[...],
                           bB_r[...], gA_r[...], gB_r[...], beA_r[...],
                           beB_r[...])
        hAs = hA * dv_ref[...]
        hBs = hB * dv_ref[...]
        pA = (jnp.dot(hAs, daa[...], preferred_element_type=jnp.float32)
              + jnp.dot(hBs, dba[...], preferred_element_type=jnp.float32))
        pB = (jnp.dot(hAs, dab[...], preferred_element_type=jnp.float32)
              + jnp.dot(hBs, dbb[...], preferred_element_type=jnp.float32))
        out_ref[0, :, :] = pA
        out_ref[1, :, :] = pB

    vec = pl.BlockSpec((1, 128), lambda i: (0, 0))
    mat = pl.BlockSpec((128, 128), lambda i: (0, 0))
    return pl.pallas_call(
        body,
        grid=(GRID,),
        in_specs=[
            pl.BlockSpec((2, PACK, 128), lambda i: (0, i, 0)),
            pl.BlockSpec((2, PACK, 128), lambda i: (0, i, 0)),
            pl.BlockSpec((PACK, 128), lambda i: (i, 0)),
            vec, vec, vec, vec, vec, vec,
            mat, mat, mat, mat,
        ],
        out_specs=pl.BlockSpec((2, PACK, 128), lambda i: (0, i, 0)),
        out_shape=jax.ShapeDtypeStruct((2, NPACK, 128), jnp.float32),
    )(acc, hws, dinv_p, bA, bB, gA, gB, beA, beB, DAA, DBA, DAB, DBB)


def _tc_output(acc, hws, dinv_p, bA, bB, gA, gB, beA, beB, nd_p, inf_p,
               w10t, w12t, bs1t, DO1A, DO1B, DS2C, bo1t, DO2, bo2t, DO3, b_o3):
    def body(acc_ref, hws_ref, dv_ref, bA_r, bB_r, gA_r, gB_r, beA_r, beB_r,
             nd_ref, inf_ref, w10_r, w12_r, bs1_r, do1a, do1b, ds2c, bo1_r,
             do2, bo2_r, do3, bo3_r, out_ref):
        hA, hB = _packed_h(acc_ref, hws_ref, dv_ref[...], bA_r[...],
                           bB_r[...], gA_r[...], gB_r[...], beA_r[...],
                           beB_r[...])
        s_pre = nd_ref[...] * w10_r[...] + inf_ref[...] * w12_r[...] + bs1_r[...]
        sr = jnp.maximum(s_pre, 0.0)
        o1 = (jnp.dot(hA, do1a[...], preferred_element_type=jnp.float32)
              + jnp.dot(hB, do1b[...], preferred_element_type=jnp.float32)
              + jnp.dot(sr, ds2c[...], preferred_element_type=jnp.float32)
              + bo1_r[...])
        o1 = jnp.maximum(o1, 0.0)
        o2 = jnp.dot(o1, do2[...], preferred_element_type=jnp.float32) + bo2_r[...]
        o2 = jnp.maximum(o2, 0.0)
        o3 = jnp.dot(o2, do3[...], preferred_element_type=jnp.float32) + bo3_r[...]
        out_ref[...] = jax.nn.sigmoid(o3)

    vec = pl.BlockSpec((1, 128), lambda i: (0, 0))
    vec256 = pl.BlockSpec((1, 256), lambda i: (0, 0))
    return pl.pallas_call(
        body,
        grid=(GRID,),
        in_specs=[
            pl.BlockSpec((2, PACK, 128), lambda i: (0, i, 0)),
            pl.BlockSpec((2, PACK, 128), lambda i: (0, i, 0)),
            pl.BlockSpec((PACK, 128), lambda i: (i, 0)),
            vec, vec, vec, vec, vec, vec,
            pl.BlockSpec((PACK, 128), lambda i: (i, 0)),
            pl.BlockSpec((PACK, 128), lambda i: (i, 0)),
            vec, vec, vec,
            pl.BlockSpec((128, 256), lambda i: (0, 0)),
            pl.BlockSpec((128, 256), lambda i: (0, 0)),
            pl.BlockSpec((128, 256), lambda i: (0, 0)),
            vec256,
            pl.BlockSpec((256, 128), lambda i: (0, 0)),
            vec,
            pl.BlockSpec((128, 4), lambda i: (0, 0)),
            pl.BlockSpec((1, 1), lambda i: (0, 0)),
        ],
        out_specs=pl.BlockSpec((PACK, 4), lambda i: (i, 0)),
        out_shape=jax.ShapeDtypeStruct((NPACK, 4), jnp.float32),
    )(acc, hws, dinv_p, bA, bB, gA, gB, beA, beB, nd_p, inf_p,
      w10t, w12t, bs1t, DO1A, DO1B, DS2C, bo1t, DO2, bo2t, DO3, b_o3)


def _tile4(v):
    return jnp.tile(v, 4).reshape(1, -1)


def _d4(M):
    return jnp.kron(jnp.eye(4, dtype=jnp.float32), M)


def _pack32(col):
    return jnp.broadcast_to(col, (N, HD)).reshape(NPACK, 128)


def kernel(x, edge_index, W_in, b_in, W_g0, b_g0, W_g1, b_g1, W_g2, b_g2,
           bn_gamma, bn_beta, W_s1, b_s1, W_s2, b_s2, W_o1, b_o1, W_o2, b_o2,
           W_o3, b_o3):
    ei_t = edge_index.reshape(2, ALL_CHUNKS, FCHUNK).transpose(1, 0, 2)
    din_parts, dout_parts = _sc_degrees(ei_t)
    dinv_row, degf_row = _tc_prep(din_parts, dout_parts)
    degf = degf_row.reshape(NPAD)
    infl_parts = _sc_influence(ei_t, degf)
    nd_row, inf_row = _tc_structural(degf_row, infl_parts)

    dinv_col = dinv_row.reshape(NPAD, 1)[:N]
    dinv_p = _pack32(dinv_col)
    nd_p = _pack32(nd_row.reshape(NPAD, 1)[:N])
    inf_p = _pack32(inf_row.reshape(NPAD, 1)[:N])

    b_in2 = b_in.reshape(1, DH)

    def halves(v):
        return _tile4(v[:HD]), _tile4(v[HD:])

    bA = [None] * 3
    bB = [None] * 3
    gA = [None] * 3
    gB = [None] * 3
    beA = [None] * 3
    beB = [None] * 3
    for l, bb in enumerate((b_g0, b_g1, b_g2)):
        bA[l], bB[l] = halves(bb)
        gA[l], gB[l] = halves(bn_gamma[l])
        beA[l], beB[l] = halves(bn_beta[l])

    def diag_quads(W):
        return (_d4(W[:HD, :HD]), _d4(W[HD:, :HD]),
                _d4(W[:HD, HD:]), _d4(W[HD:, HD:]))

    D1 = diag_quads(W_g1)
    D2 = diag_quads(W_g2)

    w10t = _tile4(W_s1[0])
    w12t = _tile4(W_s1[2])
    bs1t = _tile4(b_s1)
    DO1A = _d4(W_o1[:HD, :])
    DO1B = _d4(W_o1[HD:DH, :])
    DS2C = _d4(W_s2 @ W_o1[DH:, :])
    bo1t = _tile4(b_o1 + b_s2 @ W_o1[DH:, :])
    DO2 = _d4(W_o2)
    bo2t = _tile4(b_o2)
    DO3 = _d4(W_o3)
    bo3 = b_o3.reshape(1, 1)

    hws0 = _tc_input_layer(x, W_in, b_in2, W_g0, dinv_col)
    acc0 = _sc_scatter_features(ei_t, hws0).reshape(2, APACK, 128)
    hws0_p = hws0.reshape(2, NPACK, 128)
    hws1 = _tc_mid_layer(acc0, hws0_p, dinv_p, bA[0], bB[0], gA[0], gB[0],
                         beA[0], beB[0], *D1)
    acc1 = _sc_scatter_features(ei_t, hws1.reshape(2, N, HD)).reshape(2, APACK, 128)
    hws2 = _tc_mid_layer(acc1, hws1, dinv_p, bA[1], bB[1], gA[1], gB[1],
                         beA[1], beB[1], *D2)
    acc2 = _sc_scatter_features(ei_t, hws2.reshape(2, N, HD)).reshape(2, APACK, 128)

    out4 = _tc_output(acc2, hws2, dinv_p, bA[2], bB[2], gA[2], gB[2], beA[2],
                      beB[2], nd_p, inf_p, w10t, w12t, bs1t, DO1A, DO1B, DS2C,
                      bo1t, DO2, bo2t, DO3, bo3)
    return out4.reshape(N, 1)
